# Initial kernel scaffold; baseline (speedup 1.0000x reference)
import jax
import jax.numpy as jnp
from jax.experimental import pallas as pl


def kernel(naming_types, group_types, line_ids, naming_type_embeddings, group_type_embeddings, lines_num_embeddings):
    raise NotImplementedError("write your pallas kernel here")


# speedup vs baseline: 6.9239x; 6.9239x over previous
import functools

import jax
import jax.numpy as jnp
from jax import lax
from jax.experimental import pallas as pl
from jax.experimental.pallas import tpu as pltpu
from jax.experimental.pallas import tpu_sc as plsc


def _build_sc_kernel(N, ES, NW, C):
    per_w = N // NW
    n_chunks = per_w // C
    mesh = plsc.VectorSubcoreMesh(core_axis_name="c", subcore_axis_name="s")

    @functools.partial(
        pl.kernel,
        out_type=jax.ShapeDtypeStruct((N, ES), jnp.float32),
        mesh=mesh,
        scratch_types=[
            pltpu.VMEM((C,), jnp.int32),
            pltpu.VMEM((C,), jnp.int32),
            pltpu.VMEM((C,), jnp.int32),
            pltpu.VMEM((C, ES), jnp.float32),
            pltpu.VMEM((C, ES), jnp.float32),
            pltpu.VMEM((C, ES), jnp.float32),
            pltpu.SemaphoreType.DMA,
        ],
        compiler_params=pltpu.CompilerParams(use_tc_tiling_on_sc=False),
    )
    def k(nt, gt, li, nte, gte, lne, out, ia, ib, ic, ra, rb, rc, sem):
        cid = lax.axis_index("c")
        sid = lax.axis_index("s")
        wid = sid * 2 + cid

        def chunk(ci, carry):
            base = wid * per_w + ci * C
            pltpu.sync_copy(nt.at[pl.ds(base, C)], ia)
            pltpu.sync_copy(gt.at[pl.ds(base, C)], ib)
            pltpu.sync_copy(li.at[pl.ds(base, C)], ic)
            ca = pltpu.async_copy(nte.at[ia], ra, sem)
            cb = pltpu.async_copy(gte.at[ib], rb, sem)
            cc = pltpu.async_copy(lne.at[ic], rc, sem)
            ca.wait()
            cb.wait()
            cc.wait()

            def row(r, rcarry):
                for j in range(ES // 16):
                    s = pl.ds(j * 16, 16)
                    ra[r, s] = ra[r, s] + rb[r, s] + rc[r, s]
                return rcarry

            lax.fori_loop(0, C, row, None)
            pltpu.sync_copy(ra, out.at[pl.ds(base, C)])
            return carry

        lax.fori_loop(0, n_chunks, chunk, None)

    return k


def kernel(naming_types, group_types, line_ids, naming_type_embeddings,
           group_type_embeddings, lines_num_embeddings):
    B, L = naming_types.shape
    ES = naming_type_embeddings.shape[1]
    N = B * L
    NW = 32
    C = 512
    nt = naming_types.reshape(N).astype(jnp.int32)
    gt = group_types.reshape(N).astype(jnp.int32)
    li = line_ids.reshape(N).astype(jnp.int32)
    out = _build_sc_kernel(N, ES, NW, C)(
        nt, gt, li, naming_type_embeddings, group_type_embeddings,
        lines_num_embeddings)
    return out.reshape(B, L, ES)


# speedup vs baseline: 7.8502x; 1.1338x over previous
import functools

import jax
import jax.numpy as jnp
from jax import lax
from jax.experimental import pallas as pl
from jax.experimental.pallas import tpu as pltpu
from jax.experimental.pallas import tpu_sc as plsc


def _build_sc_kernel(N, ES, NW, C):
    per_w = N // NW
    n_chunks = per_w // C
    mesh = plsc.VectorSubcoreMesh(core_axis_name="c", subcore_axis_name="s")

    @functools.partial(
        pl.kernel,
        out_type=jax.ShapeDtypeStruct((N, ES), jnp.float32),
        mesh=mesh,
        scratch_types=[
            pltpu.VMEM((C,), jnp.int32),
            pltpu.VMEM((C,), jnp.int32),
            pltpu.VMEM((C,), jnp.int32),
            pltpu.VMEM((C, ES), jnp.float32),
            pltpu.SemaphoreType.DMA,
        ],
        compiler_params=pltpu.CompilerParams(use_tc_tiling_on_sc=False),
    )
    def k(nt, gt, li, nte, gte, lne, out, ia, ib, ic, ra, sem):
        cid = lax.axis_index("c")
        sid = lax.axis_index("s")
        wid = sid * 2 + cid

        def chunk(ci, carry):
            base = wid * per_w + ci * C
            pltpu.sync_copy(nt.at[pl.ds(base, C)], ia)
            pltpu.sync_copy(gt.at[pl.ds(base, C)], ib)
            pltpu.sync_copy(li.at[pl.ds(base, C)], ic)
            pltpu.async_copy(nte.at[ia], ra, sem).wait()
            pltpu.async_copy(gte.at[ib], ra, sem, add=True).wait()
            pltpu.async_copy(lne.at[ic], ra, sem, add=True).wait()
            pltpu.sync_copy(ra, out.at[pl.ds(base, C)])
            return carry

        lax.fori_loop(0, n_chunks, chunk, None)

    return k


def kernel(naming_types, group_types, line_ids, naming_type_embeddings,
           group_type_embeddings, lines_num_embeddings):
    B, L = naming_types.shape
    ES = naming_type_embeddings.shape[1]
    N = B * L
    NW = 32
    C = 512
    nt = naming_types.reshape(N).astype(jnp.int32)
    gt = group_types.reshape(N).astype(jnp.int32)
    li = line_ids.reshape(N).astype(jnp.int32)
    out = _build_sc_kernel(N, ES, NW, C)(
        nt, gt, li, naming_type_embeddings, group_type_embeddings,
        lines_num_embeddings)
    return out.reshape(B, L, ES)


# speedup vs baseline: 8.7460x; 1.1141x over previous
import functools

import jax
import jax.numpy as jnp
from jax import lax
from jax.experimental import pallas as pl
from jax.experimental.pallas import tpu as pltpu
from jax.experimental.pallas import tpu_sc as plsc

_P = 4


def _build_sc_kernel(N, ES, NW, C):
    per_w = N // NW
    n_chunks = per_w // C
    P = _P
    assert n_chunks % P == 0 and n_chunks >= 2 * P
    G = (n_chunks - P) // P
    mesh = plsc.VectorSubcoreMesh(core_axis_name="c", subcore_axis_name="s")
    scratch = (
        [pltpu.VMEM((C, ES), jnp.float32)] * P
        + [pltpu.VMEM((C,), jnp.int32)] * (3 * P)
        + [pltpu.SemaphoreType.DMA] * (3 * P)
    )

    @functools.partial(
        pl.kernel,
        out_type=jax.ShapeDtypeStruct((N, ES), jnp.float32),
        mesh=mesh,
        scratch_types=scratch,
        compiler_params=pltpu.CompilerParams(use_tc_tiling_on_sc=False),
    )
    def k(nt, gt, li, nte, gte, lne, out, *scr):
        rows = scr[0:P]
        idxs = [scr[P + 3 * b: P + 3 * b + 3] for b in range(P)]
        sga = scr[4 * P: 5 * P]
        sbc = scr[5 * P: 6 * P]
        ss = scr[6 * P: 7 * P]
        wid = lax.axis_index("s") * 2 + lax.axis_index("c")
        wbase = wid * per_w

        def scat_wait(b, base):
            pltpu.make_async_copy(rows[b], out.at[pl.ds(base, C)], ss[b]).wait()

        def stage_a(b, base, wait_base=None):
            if wait_base is not None:
                scat_wait(b, wait_base)
            pltpu.sync_copy(nt.at[pl.ds(base, C)], idxs[b][0])
            pltpu.sync_copy(gt.at[pl.ds(base, C)], idxs[b][1])
            pltpu.sync_copy(li.at[pl.ds(base, C)], idxs[b][2])
            pltpu.async_copy(nte.at[idxs[b][0]], rows[b], sga[b])

        def stage_b(b):
            pltpu.make_async_copy(nte.at[idxs[b][0]], rows[b], sga[b]).wait()
            pltpu.async_copy(gte.at[idxs[b][1]], rows[b], sbc[b], add=True)
            pltpu.async_copy(lne.at[idxs[b][2]], rows[b], sbc[b], add=True)

        def stage_c(b, base):
            pltpu.make_async_copy(gte.at[idxs[b][1]], rows[b], sbc[b]).wait()
            pltpu.make_async_copy(lne.at[idxs[b][2]], rows[b], sbc[b]).wait()
            pltpu.async_copy(rows[b], out.at[pl.ds(base, C)], ss[b])

        stage_a(0, wbase + 0 * C)
        stage_a(1, wbase + 1 * C)
        stage_b(0)
        stage_a(2, wbase + 2 * C)
        stage_b(1)
        stage_c(0, wbase + 0 * C)
        stage_a(3, wbase + 3 * C)
        stage_b(2)
        stage_c(1, wbase + 1 * C)

        def group(g, carry):
            ci0 = P + P * g
            for b in range(P):
                base = wbase + (ci0 + b) * C
                stage_a(b, base, wait_base=base - P * C)
                stage_b((b + P - 1) % P)
                stage_c((b + P - 2) % P, base - 2 * C)
            return carry

        lax.fori_loop(0, G, group, None)

        last = n_chunks - 1
        stage_b(last % P)
        stage_c((last - 1) % P, wbase + (last - 1) * C)
        stage_c(last % P, wbase + last * C)
        for m in range(n_chunks - P, n_chunks):
            scat_wait(m % P, wbase + m * C)

    return k


def kernel(naming_types, group_types, line_ids, naming_type_embeddings,
           group_type_embeddings, lines_num_embeddings):
    B, L = naming_types.shape
    ES = naming_type_embeddings.shape[1]
    N = B * L
    NW = 32
    C = 400
    nt = naming_types.reshape(N).astype(jnp.int32)
    gt = group_types.reshape(N).astype(jnp.int32)
    li = line_ids.reshape(N).astype(jnp.int32)
    out = _build_sc_kernel(N, ES, NW, C)(
        nt, gt, li, naming_type_embeddings, group_type_embeddings,
        lines_num_embeddings)
    return out.reshape(B, L, ES)
